# baseline (speedup 1.0000x reference)
import functools

import jax
import jax.numpy as jnp
from jax import lax
from jax.experimental import pallas as pl
from jax.experimental.pallas import tpu as pltpu
from jax.experimental.pallas import tpu_sc as plsc

_V = 1000000
_D = 64
_B = 16384
_NEG = 5

_NC = 2
_NS = 16
_NW = _NC * _NS
_L = 16

_CHUNK = _B // _NW
_NCH = _CHUNK // _L


def _sc_scores(u3, v3, pos_u, pos_v, negf):
    mesh = plsc.VectorSubcoreMesh(
        core_axis_name="c", subcore_axis_name="s",
        num_cores=_NC, num_subcores=_NS)

    @functools.partial(
        pl.kernel,
        out_type=(
            jax.ShapeDtypeStruct((_B,), jnp.float32),
            jax.ShapeDtypeStruct((_NEG * _B,), jnp.float32),
        ),
        mesh=mesh,
        scratch_types=[
            pltpu.VMEM((_CHUNK,), jnp.int32),
            pltpu.VMEM((_CHUNK,), jnp.int32),
            pltpu.VMEM((_CHUNK * _NEG,), jnp.int32),
            pltpu.VMEM((_L * 8, _D), jnp.float32),
            pltpu.VMEM((_L * 8, _D), jnp.float32),
            pltpu.VMEM((_L * _NEG * 8, _D), jnp.float32),
            pltpu.VMEM((_CHUNK,), jnp.float32),
            [pltpu.VMEM((_CHUNK,), jnp.float32) for _ in range(_NEG)],
            pltpu.SemaphoreType.DMA,
        ],
        compiler_params=pltpu.CompilerParams(needs_layout_passes=False),
    )
    def k(u_hbm, v_hbm, pu_hbm, pv_hbm, nv_hbm, pos_out, neg_out,
          idxu, idxv, idxn, utile, vtile, ntile, psc, nsc, sem):
        wid = lax.axis_index("s") * _NC + lax.axis_index("c")
        base = wid * _CHUNK
        pltpu.sync_copy(pu_hbm.at[pl.ds(base, _CHUNK)], idxu)
        pltpu.sync_copy(pv_hbm.at[pl.ds(base, _CHUNK)], idxv)
        pltpu.sync_copy(nv_hbm.at[pl.ds(base * _NEG, _CHUNK * _NEG)], idxn)
        iota = lax.iota(jnp.int32, _L)

        def chunk_body(c, carry):
            off = c * _L
            iu = idxu[pl.ds(off, _L)]
            iv = idxv[pl.ds(off, _L)]
            bu = jnp.bitwise_and(iu, -8)
            bv = jnp.bitwise_and(iv, -8)
            inj = [idxn[pl.ds(off * _NEG + _L * j, _L)] for j in range(_NEG)]
            bn = [jnp.bitwise_and(inj[j], -8) for j in range(_NEG)]
            cps = []
            for t in range(_L):
                cps.append(pltpu.async_copy(
                    u_hbm.at[pl.ds(pl.multiple_of(bu[t], 8), 8)],
                    utile.at[pl.ds(8 * t, 8)], sem))
                cps.append(pltpu.async_copy(
                    v_hbm.at[pl.ds(pl.multiple_of(bv[t], 8), 8)],
                    vtile.at[pl.ds(8 * t, 8)], sem))
                for j in range(_NEG):
                    cps.append(pltpu.async_copy(
                        v_hbm.at[pl.ds(pl.multiple_of(bn[j][t], 8), 8)],
                        ntile.at[pl.ds(8 * (_L * j + t), 8)], sem))
            subu = jnp.bitwise_and(iu, 7)
            subv = jnp.bitwise_and(iv, 7)
            rown = [iota * _NEG + n for n in range(_NEG)]
            subn = [
                jnp.bitwise_and(plsc.load_gather(idxn, [off * _NEG + rn]), 7)
                for rn in rown
            ]
            for cp in cps:
                cp.wait()

            def dbody(dd, acc):
                col = jnp.full((_L,), dd, jnp.int32)
                uval = plsc.load_gather(utile, [iota * 8 + subu, col])
                vval = plsc.load_gather(vtile, [iota * 8 + subv, col])
                new0 = acc[0] + uval * vval
                rest = tuple(
                    acc[1 + n]
                    + uval * plsc.load_gather(
                        ntile, [rown[n] * 8 + subn[n], col])
                    for n in range(_NEG))
                return (new0,) + rest

            z = jnp.zeros((_L,), jnp.float32)
            accs = lax.fori_loop(0, _D, dbody, (z,) * (1 + _NEG))
            psc[pl.ds(off, _L)] = accs[0]
            for n in range(_NEG):
                nsc[n][pl.ds(off, _L)] = accs[1 + n]
            return carry

        lax.fori_loop(0, _NCH, chunk_body, 0)

        pltpu.sync_copy(psc, pos_out.at[pl.ds(base, _CHUNK)])
        for n in range(_NEG):
            pltpu.sync_copy(nsc[n],
                            neg_out.at[pl.ds(n * _B + base, _CHUNK)])

    return k(u3, v3, pos_u, pos_v, negf)


def _loss_body(p_ref, n_ref, o_ref):
    s = jnp.clip(p_ref[...], -10.0, 10.0)
    t1 = jnp.sum(jnp.log(1.0 + jnp.exp(-s)))
    ns = jnp.clip(n_ref[...], -10.0, 10.0)
    t2 = jnp.sum(jnp.log(1.0 + jnp.exp(ns)))
    o_ref[...] = jnp.reshape((t1 + t2) * (1.0 / _B), (1, 1))


def _loss_tc(pos_sc, neg_sc):
    out = pl.pallas_call(
        _loss_body,
        out_shape=jax.ShapeDtypeStruct((1, 1), jnp.float32),
    )(pos_sc.reshape(_B // 128, 128), neg_sc.reshape(_B * _NEG // 128, 128))
    return out[0, 0]


def kernel(u_emb, v_emb, pos_u, pos_v, neg_v):
    negf = neg_v.reshape(_B * _NEG)
    pos_sc, neg_sc = _sc_scores(u_emb, v_emb, pos_u, pos_v, negf)
    return _loss_tc(pos_sc, neg_sc)


# speedup vs baseline: 1.3460x; 1.3460x over previous
import functools

import jax
import jax.numpy as jnp
from jax import lax
from jax.experimental import pallas as pl
from jax.experimental.pallas import tpu as pltpu
from jax.experimental.pallas import tpu_sc as plsc

_V = 1000000
_D = 64
_B = 16384
_NEG = 5

_NC = 2
_NS = 16
_NW = _NC * _NS
_L = 16

_CHUNK = _B // _NW
_NCH = _CHUNK // _L


def _sc_scores(u3, v3, pos_u, pos_v, negf):
    mesh = plsc.VectorSubcoreMesh(
        core_axis_name="c", subcore_axis_name="s",
        num_cores=_NC, num_subcores=_NS)

    @functools.partial(
        pl.kernel,
        out_type=(
            jax.ShapeDtypeStruct((_B,), jnp.float32),
            jax.ShapeDtypeStruct((_NEG * _B,), jnp.float32),
        ),
        mesh=mesh,
        scratch_types=[
            pltpu.VMEM((_CHUNK,), jnp.int32),
            pltpu.VMEM((_CHUNK,), jnp.int32),
            pltpu.VMEM((_CHUNK * _NEG,), jnp.int32),
            pltpu.VMEM((_L, 8, _D), jnp.float32),
            pltpu.VMEM((_L, 8, _D), jnp.float32),
            pltpu.VMEM((_L * _NEG, 8, _D), jnp.float32),
            pltpu.VMEM((_CHUNK,), jnp.float32),
            [pltpu.VMEM((_CHUNK,), jnp.float32) for _ in range(_NEG)],
            pltpu.SemaphoreType.DMA,
        ],
        compiler_params=pltpu.CompilerParams(needs_layout_passes=False),
    )
    def k(u_hbm, v_hbm, pu_hbm, pv_hbm, nv_hbm, pos_out, neg_out,
          idxu, idxv, idxn, utile, vtile, ntile, psc, nsc, sem):
        wid = lax.axis_index("s") * _NC + lax.axis_index("c")
        base = wid * _CHUNK
        pltpu.sync_copy(pu_hbm.at[pl.ds(base, _CHUNK)], idxu)
        pltpu.sync_copy(pv_hbm.at[pl.ds(base, _CHUNK)], idxv)
        pltpu.sync_copy(nv_hbm.at[pl.ds(base * _NEG, _CHUNK * _NEG)], idxn)
        iota = lax.iota(jnp.int32, _L)

        def chunk_body(c, carry):
            off = c * _L
            iu = idxu[pl.ds(off, _L)]
            iv = idxv[pl.ds(off, _L)]
            tu = lax.shift_right_logical(iu, 4)
            tv = lax.shift_right_logical(iv, 4)
            hu = jnp.bitwise_and(iu, 8)
            hv = jnp.bitwise_and(iv, 8)
            inj = [idxn[pl.ds(off * _NEG + _L * j, _L)] for j in range(_NEG)]
            tn = [lax.shift_right_logical(inj[j], 4) for j in range(_NEG)]
            hn = [jnp.bitwise_and(inj[j], 8) for j in range(_NEG)]
            cps = []
            for t in range(_L):
                cps.append(pltpu.async_copy(
                    u_hbm.at[pl.ds(tu[t], 1),
                             pl.ds(pl.multiple_of(hu[t], 8), 8)],
                    utile.at[pl.ds(t, 1)], sem))
                cps.append(pltpu.async_copy(
                    v_hbm.at[pl.ds(tv[t], 1),
                             pl.ds(pl.multiple_of(hv[t], 8), 8)],
                    vtile.at[pl.ds(t, 1)], sem))
                for j in range(_NEG):
                    cps.append(pltpu.async_copy(
                        v_hbm.at[pl.ds(tn[j][t], 1),
                                 pl.ds(pl.multiple_of(hn[j][t], 8), 8)],
                        ntile.at[pl.ds(_L * j + t, 1)], sem))
            subu = jnp.bitwise_and(iu, 7)
            subv = jnp.bitwise_and(iv, 7)
            rown = [iota * _NEG + n for n in range(_NEG)]
            subn = [
                jnp.bitwise_and(plsc.load_gather(idxn, [off * _NEG + rn]), 7)
                for rn in rown
            ]
            for cp in cps:
                cp.wait()

            def dbody(dd, acc):
                col = jnp.full((_L,), dd, jnp.int32)
                uval = plsc.load_gather(utile, [iota, subu, col])
                vval = plsc.load_gather(vtile, [iota, subv, col])
                new0 = acc[0] + uval * vval
                rest = tuple(
                    acc[1 + n]
                    + uval * plsc.load_gather(
                        ntile, [rown[n], subn[n], col])
                    for n in range(_NEG))
                return (new0,) + rest

            z = jnp.zeros((_L,), jnp.float32)
            accs = lax.fori_loop(0, _D, dbody, (z,) * (1 + _NEG))
            psc[pl.ds(off, _L)] = accs[0]
            for n in range(_NEG):
                nsc[n][pl.ds(off, _L)] = accs[1 + n]
            return carry

        lax.fori_loop(0, _NCH, chunk_body, 0)

        pltpu.sync_copy(psc, pos_out.at[pl.ds(base, _CHUNK)])
        for n in range(_NEG):
            pltpu.sync_copy(nsc[n],
                            neg_out.at[pl.ds(n * _B + base, _CHUNK)])

    return k(u3, v3, pos_u, pos_v, negf)


def _loss_body(p_ref, n_ref, o_ref):
    s = jnp.clip(p_ref[...], -10.0, 10.0)
    t1 = jnp.sum(jnp.log(1.0 + jnp.exp(-s)))
    ns = jnp.clip(n_ref[...], -10.0, 10.0)
    t2 = jnp.sum(jnp.log(1.0 + jnp.exp(ns)))
    o_ref[...] = jnp.reshape((t1 + t2) * (1.0 / _B), (1, 1))


def _loss_tc(pos_sc, neg_sc):
    out = pl.pallas_call(
        _loss_body,
        out_shape=jax.ShapeDtypeStruct((1, 1), jnp.float32),
    )(pos_sc.reshape(_B // 128, 128), neg_sc.reshape(_B * _NEG // 128, 128))
    return out[0, 0]


def kernel(u_emb, v_emb, pos_u, pos_v, neg_v):
    u3 = u_emb.reshape(_V // 16, 16, _D)
    v3 = v_emb.reshape(_V // 16, 16, _D)
    negf = neg_v.reshape(_B * _NEG)
    pos_sc, neg_sc = _sc_scores(u3, v3, pos_u, pos_v, negf)
    return _loss_tc(pos_sc, neg_sc)
